# baseline (speedup 1.0000x reference)
import functools

import jax
import jax.numpy as jnp
from jax.experimental import pallas as pl
from jax.experimental.pallas import tpu as pltpu

S, D, E, F, K = 2048, 1024, 8, 2048, 2
T = 256
NB = (S * K) // T + E
R = NB * T


def _router_body(x_ref, wr_ref, idx_ref, gate_ref):
    x = x_ref[...]
    logits = jnp.dot(x, wr_ref[...], preferred_element_type=jnp.float32)
    lane = jax.lax.broadcasted_iota(jnp.int32, (S, E), 1)
    m1 = jnp.max(logits, axis=1, keepdims=True)
    i1 = jnp.argmax(logits, axis=1)[:, None]
    masked = jnp.where(lane == i1, -jnp.inf, logits)
    m2 = jnp.max(masked, axis=1, keepdims=True)
    i2 = jnp.argmax(masked, axis=1)[:, None]
    d = jnp.exp(m2 - m1)
    g1 = 1.0 / (1.0 + d)
    g2 = d / (1.0 + d)
    idx_ref[...] = jnp.concatenate([i1, i2], axis=1).astype(jnp.int32)
    gate_ref[...] = jnp.concatenate([g1, g2], axis=1)


def _router(x2d, wr):
    return pl.pallas_call(
        _router_body,
        out_shape=(
            jax.ShapeDtypeStruct((S, K), jnp.int32),
            jax.ShapeDtypeStruct((S, K), jnp.float32),
        ),
    )(x2d, wr)


def _ffn_body(be_ref, x_ref, w1_ref, b1_ref, w2_ref, b2_ref, o_ref):
    del be_ref
    h = jnp.dot(x_ref[...], w1_ref[0], preferred_element_type=jnp.float32)
    h = jax.nn.gelu(h + b1_ref[0]).astype(jnp.bfloat16)
    y = jnp.dot(h, w2_ref[0], preferred_element_type=jnp.float32)
    o_ref[...] = y + b2_ref[0]


def _ffn(x_sorted, W1, b1, W2, b2, block_expert):
    grid_spec = pltpu.PrefetchScalarGridSpec(
        num_scalar_prefetch=1,
        grid=(NB,),
        in_specs=[
            pl.BlockSpec((T, D), lambda b, be: (b, 0)),
            pl.BlockSpec((1, D, F), lambda b, be: (be[b], 0, 0)),
            pl.BlockSpec((1, 1, F), lambda b, be: (be[b], 0, 0)),
            pl.BlockSpec((1, F, D), lambda b, be: (be[b], 0, 0)),
            pl.BlockSpec((1, 1, D), lambda b, be: (be[b], 0, 0)),
        ],
        out_specs=pl.BlockSpec((T, D), lambda b, be: (b, 0)),
    )
    return pl.pallas_call(
        _ffn_body,
        grid_spec=grid_spec,
        out_shape=jax.ShapeDtypeStruct((R, D), jnp.float32),
    )(block_expert, x_sorted, W1.astype(jnp.bfloat16),
      b1.reshape(E, 1, F), W2.astype(jnp.bfloat16), b2.reshape(E, 1, D))


def kernel(x, W_router, W1, b1, W2, b2):
    x2d = x.reshape(S, D)
    idx, gates = _router(x2d, W_router)

    pairs_e = idx.reshape(S * K)
    onehot = (pairs_e[:, None] == jnp.arange(E, dtype=jnp.int32)[None, :])
    cum = jnp.cumsum(onehot.astype(jnp.int32), axis=0)
    counts = cum[-1]
    nblk = (counts + T - 1) // T
    blk_base = jnp.concatenate(
        [jnp.zeros((1,), jnp.int32), jnp.cumsum(nblk)[:-1].astype(jnp.int32)])
    rank = jnp.take_along_axis(cum, pairs_e[:, None], axis=1)[:, 0] - 1
    pos = blk_base[pairs_e] * T + rank
    bids = jnp.arange(NB, dtype=jnp.int32)
    block_expert = (
        jnp.sum((blk_base[None, :] <= bids[:, None]).astype(jnp.int32), axis=1)
        - 1).astype(jnp.int32)

    tok = jnp.arange(S * K, dtype=jnp.int32) // K
    sorted_tok = jnp.zeros((R,), jnp.int32).at[pos].set(tok)
    x_sorted = x2d.astype(jnp.bfloat16)[sorted_tok]

    y_sorted = _ffn(x_sorted, W1, b1, W2, b2, block_expert)

    y2 = y_sorted[pos].reshape(S, K, D)
    out = jnp.sum(gates[:, :, None] * y2, axis=1)
    return out.reshape(1, S, D)


# speedup vs baseline: 1.2637x; 1.2637x over previous
import functools

import jax
import jax.numpy as jnp
from jax import lax
from jax.experimental import pallas as pl
from jax.experimental.pallas import tpu as pltpu
from jax.experimental.pallas import tpu_sc as plsc

S, D, E, F, K = 2048, 1024, 8, 2048, 2
T = 256
NB = (S * K) // T + E
R = NB * T

NW = 32
PPW = (S * K) // NW
CH = 32
NCH = PPW // CH
TPW = S // NW


def _router_body(x_ref, wr_ref, idx_ref, gate_ref):
    x = x_ref[...]
    logits = jnp.dot(x, wr_ref[...], preferred_element_type=jnp.float32)
    lane = lax.broadcasted_iota(jnp.int32, (S, E), 1)
    m1 = jnp.max(logits, axis=1, keepdims=True)
    i1 = jnp.argmax(logits, axis=1)[:, None]
    masked = jnp.where(lane == i1, -jnp.inf, logits)
    m2 = jnp.max(masked, axis=1, keepdims=True)
    i2 = jnp.argmax(masked, axis=1)[:, None]
    d = jnp.exp(m2 - m1)
    g1 = 1.0 / (1.0 + d)
    g2 = d / (1.0 + d)
    idx_ref[...] = jnp.concatenate([i1, i2], axis=1).astype(jnp.int32)
    gate_ref[...] = jnp.concatenate([g1, g2], axis=1)


def _router(x2d, wr):
    return pl.pallas_call(
        _router_body,
        out_shape=(
            jax.ShapeDtypeStruct((S, K), jnp.int32),
            jax.ShapeDtypeStruct((S, K), jnp.float32),
        ),
    )(x2d, wr)


def _dispatch_body(x_hbm, tok_hbm, pos_hbm, xs_hbm,
                   idx_t, idx_p, rows, sem_g, sem_s):
    wid = lax.axis_index("s") * 2 + lax.axis_index("c")
    base = wid * PPW
    for c in range(NCH):
        off = base + c * CH
        pltpu.sync_copy(tok_hbm.at[pl.ds(off, CH)], idx_t)
        pltpu.sync_copy(pos_hbm.at[pl.ds(off, CH)], idx_p)
        pltpu.async_copy(x_hbm.at[idx_t], rows, sem_g).wait()
        pltpu.async_copy(rows, xs_hbm.at[idx_p], sem_s).wait()


def _dispatch(x2d, tok, pos):
    mesh = plsc.VectorSubcoreMesh(core_axis_name="c", subcore_axis_name="s")
    return pl.kernel(
        _dispatch_body,
        out_type=jax.ShapeDtypeStruct((R, D), jnp.float32),
        mesh=mesh,
        scratch_types=[
            pltpu.VMEM((CH,), jnp.int32),
            pltpu.VMEM((CH,), jnp.int32),
            pltpu.VMEM((CH, D), jnp.float32),
            pltpu.SemaphoreType.DMA,
            pltpu.SemaphoreType.DMA,
        ],
    )(x2d, tok, pos)


def _ffn_body(be_ref, x_ref, w1_ref, b1_ref, w2_ref, b2_ref, gate_ref, o_ref):
    del be_ref
    xb = x_ref[...].astype(jnp.bfloat16)
    h = jnp.dot(xb, w1_ref[0], preferred_element_type=jnp.float32)
    h = jax.nn.gelu(h + b1_ref[0]).astype(jnp.bfloat16)
    y = jnp.dot(h, w2_ref[0], preferred_element_type=jnp.float32)
    o_ref[...] = (y + b2_ref[0]) * gate_ref[...]


def _ffn(x_sorted, W1, b1, W2, b2, gate_sorted, block_expert):
    grid_spec = pltpu.PrefetchScalarGridSpec(
        num_scalar_prefetch=1,
        grid=(NB,),
        in_specs=[
            pl.BlockSpec((T, D), lambda b, be: (b, 0)),
            pl.BlockSpec((1, D, F), lambda b, be: (be[b], 0, 0)),
            pl.BlockSpec((1, 1, F), lambda b, be: (be[b], 0, 0)),
            pl.BlockSpec((1, F, D), lambda b, be: (be[b], 0, 0)),
            pl.BlockSpec((1, 1, D), lambda b, be: (be[b], 0, 0)),
            pl.BlockSpec((T, 1), lambda b, be: (b, 0)),
        ],
        out_specs=pl.BlockSpec((T, D), lambda b, be: (b, 0)),
    )
    return pl.pallas_call(
        _ffn_body,
        grid_spec=grid_spec,
        out_shape=jax.ShapeDtypeStruct((R, D), jnp.float32),
    )(block_expert, x_sorted, W1.astype(jnp.bfloat16), b1.reshape(E, 1, F),
      W2.astype(jnp.bfloat16), b2.reshape(E, 1, D), gate_sorted)


def _combine_body(yg_hbm, pos_hbm, out_hbm, idx_p, rows, obuf, sem_g):
    wid = lax.axis_index("s") * 2 + lax.axis_index("c")
    for c in range(NCH):
        poff = wid * PPW + c * CH
        toff = wid * TPW + c * (CH // 2)
        pltpu.sync_copy(pos_hbm.at[pl.ds(poff, CH)], idx_p)
        pltpu.async_copy(yg_hbm.at[idx_p], rows, sem_g).wait()

        def body(j, _):
            for i in range(CH // 2):
                a = rows[2 * i, pl.ds(j * 16, 16)]
                b = rows[2 * i + 1, pl.ds(j * 16, 16)]
                obuf[i, pl.ds(j * 16, 16)] = a + b
            return 0

        lax.fori_loop(0, D // 16, body, 0)
        pltpu.sync_copy(obuf, out_hbm.at[pl.ds(toff, CH // 2)])


def _combine(yg, pos):
    mesh = plsc.VectorSubcoreMesh(core_axis_name="c", subcore_axis_name="s")
    return pl.kernel(
        _combine_body,
        out_type=jax.ShapeDtypeStruct((S, D), jnp.float32),
        mesh=mesh,
        scratch_types=[
            pltpu.VMEM((CH,), jnp.int32),
            pltpu.VMEM((CH, D), jnp.float32),
            pltpu.VMEM((CH // 2, D), jnp.float32),
            pltpu.SemaphoreType.DMA,
        ],
    )(yg, pos)


def kernel(x, W_router, W1, b1, W2, b2):
    x2d = x.reshape(S, D)
    idx, gates = _router(x2d, W_router)

    pairs_e = idx.reshape(S * K)
    onehot = (pairs_e[:, None] == jnp.arange(E, dtype=jnp.int32)[None, :])
    cum = jnp.cumsum(onehot.astype(jnp.int32), axis=0)
    counts = cum[-1]
    nblk = (counts + T - 1) // T
    blk_base = jnp.concatenate(
        [jnp.zeros((1,), jnp.int32), jnp.cumsum(nblk)[:-1].astype(jnp.int32)])
    rank = jnp.take_along_axis(cum, pairs_e[:, None], axis=1)[:, 0] - 1
    pos = blk_base[pairs_e] * T + rank
    bids = jnp.arange(NB, dtype=jnp.int32)
    block_expert = (
        jnp.sum((blk_base[None, :] <= bids[:, None]).astype(jnp.int32), axis=1)
        - 1).astype(jnp.int32)

    tok = jnp.arange(S * K, dtype=jnp.int32) // K
    gate_sorted = jnp.zeros((R, 1), jnp.float32).at[pos, 0].set(
        gates.reshape(S * K))

    x_sorted = _dispatch(x2d, tok, pos)
    yg = _ffn(x_sorted, W1, b1, W2, b2, gate_sorted, block_expert)
    out = _combine(yg, pos)
    return out.reshape(1, S, D)


# speedup vs baseline: 1.2920x; 1.0224x over previous
import functools

import jax
import jax.numpy as jnp
from jax import lax
from jax.experimental import pallas as pl
from jax.experimental.pallas import tpu as pltpu
from jax.experimental.pallas import tpu_sc as plsc

S, D, E, F, K = 2048, 1024, 8, 2048, 2
T = 128
NB = (S * K) // T + E
R = NB * T

NW = 32
PPW = (S * K) // NW
CH = 32
NCH = PPW // CH
TPW = S // NW
L = 16


def _router_body(x_ref, wr_ref, idx_ref, gate_ref):
    x = x_ref[...]
    logits = jnp.dot(x, wr_ref[...], preferred_element_type=jnp.float32)
    lane = lax.broadcasted_iota(jnp.int32, (S, E), 1)
    m1 = jnp.max(logits, axis=1, keepdims=True)
    i1 = jnp.argmax(logits, axis=1)[:, None]
    masked = jnp.where(lane == i1, -jnp.inf, logits)
    m2 = jnp.max(masked, axis=1, keepdims=True)
    i2 = jnp.argmax(masked, axis=1)[:, None]
    d = jnp.exp(m2 - m1)
    g1 = 1.0 / (1.0 + d)
    g2 = d / (1.0 + d)
    idx_ref[...] = jnp.concatenate([i1, i2], axis=1).astype(jnp.int32)
    gate_ref[...] = jnp.concatenate([g1, g2], axis=1)


def _router(x2d, wr):
    return pl.pallas_call(
        _router_body,
        out_shape=(
            jax.ShapeDtypeStruct((S, K), jnp.int32),
            jax.ShapeDtypeStruct((S, K), jnp.float32),
        ),
    )(x2d, wr)


def _dispatch_body(x_hbm, tok_hbm, pos_hbm, xs_hbm,
                   idx_t, idx_p, rows_a, rows_b, sg_a, sg_b, ss_a, ss_b):
    wid = lax.axis_index("s") * 2 + lax.axis_index("c")
    pltpu.sync_copy(tok_hbm.at[wid], idx_t)
    pltpu.sync_copy(pos_hbm.at[wid], idx_p)
    rows = (rows_a, rows_b)
    sg = (sg_a, sg_b)
    ss = (ss_a, ss_b)
    h_g = [None, None]
    h_s = [None, None]
    h_g[0] = pltpu.async_copy(x_hbm.at[idx_t.at[0]], rows[0], sg[0])
    for c in range(NCH):
        sl = c % 2
        if c + 1 < NCH:
            nsl = 1 - sl
            if h_s[nsl] is not None:
                h_s[nsl].wait()
            h_g[nsl] = pltpu.async_copy(
                x_hbm.at[idx_t.at[c + 1]], rows[nsl], sg[nsl])
        h_g[sl].wait()
        h_s[sl] = pltpu.async_copy(rows[sl], xs_hbm.at[idx_p.at[c]], ss[sl])
    h_s[0].wait()
    h_s[1].wait()


def _dispatch(x2d, tok3, pos3):
    mesh = plsc.VectorSubcoreMesh(core_axis_name="c", subcore_axis_name="s")
    return pl.kernel(
        _dispatch_body,
        out_type=jax.ShapeDtypeStruct((R, D), jnp.float32),
        mesh=mesh,
        scratch_types=[
            pltpu.VMEM((NCH, CH), jnp.int32),
            pltpu.VMEM((NCH, CH), jnp.int32),
            pltpu.VMEM((CH, D), jnp.float32),
            pltpu.VMEM((CH, D), jnp.float32),
            pltpu.SemaphoreType.DMA,
            pltpu.SemaphoreType.DMA,
            pltpu.SemaphoreType.DMA,
            pltpu.SemaphoreType.DMA,
        ],
    )(x2d, tok3, pos3)


def _ffn_body(be_ref, bv_ref, x_ref, w1_ref, b1_ref, w2_ref, b2_ref, o_ref):
    del be_ref
    b = pl.program_id(0)

    @pl.when(bv_ref[b] > 0)
    def _():
        xb = x_ref[...].astype(jnp.bfloat16)
        h = jnp.dot(xb, w1_ref[0], preferred_element_type=jnp.float32)
        h = jax.nn.gelu(h + b1_ref[0]).astype(jnp.bfloat16)
        y = jnp.dot(h, w2_ref[0], preferred_element_type=jnp.float32)
        o_ref[...] = y + b2_ref[0]


def _ffn(x_sorted, W1, b1, W2, b2, block_expert, block_valid):
    grid_spec = pltpu.PrefetchScalarGridSpec(
        num_scalar_prefetch=2,
        grid=(NB,),
        in_specs=[
            pl.BlockSpec((T, D), lambda b, be, bv: (b, 0)),
            pl.BlockSpec((1, D, F), lambda b, be, bv: (be[b], 0, 0)),
            pl.BlockSpec((1, 1, F), lambda b, be, bv: (be[b], 0, 0)),
            pl.BlockSpec((1, F, D), lambda b, be, bv: (be[b], 0, 0)),
            pl.BlockSpec((1, 1, D), lambda b, be, bv: (be[b], 0, 0)),
        ],
        out_specs=pl.BlockSpec((T, D), lambda b, be, bv: (b, 0)),
    )
    return pl.pallas_call(
        _ffn_body,
        grid_spec=grid_spec,
        out_shape=jax.ShapeDtypeStruct((R, D), jnp.float32),
    )(block_expert, block_valid, x_sorted, W1.astype(jnp.bfloat16),
      b1.reshape(E, 1, F), W2.astype(jnp.bfloat16), b2.reshape(E, 1, D))


def _combine_body(y_hbm, pos_hbm, g_hbm, out_hbm,
                  idx_p, gall, rows_a, rows_b, obuf, sg_a, sg_b):
    wid = lax.axis_index("s") * 2 + lax.axis_index("c")
    pltpu.sync_copy(pos_hbm.at[wid], idx_p)
    pltpu.sync_copy(g_hbm.at[pl.ds(wid * PPW, PPW)], gall)
    rows = (rows_a, rows_b)
    sg = (sg_a, sg_b)
    h_g = [None, None]
    h_g[0] = pltpu.async_copy(y_hbm.at[idx_p.at[0]], rows[0], sg[0])
    for c in range(NCH):
        sl = c % 2
        if c + 1 < NCH:
            h_g[1 - sl] = pltpu.async_copy(
                y_hbm.at[idx_p.at[c + 1]], rows[1 - sl], sg[1 - sl])
        h_g[sl].wait()
        rbuf = rows[sl]
        for i in range(CH // 2):
            p0 = c * CH + 2 * i
            gv = gall[pl.ds((p0 // L) * L, L)]
            ga = gv[p0 % L]
            gb = gv[p0 % L + 1]

            def body(j, _, i=i, ga=ga, gb=gb, rbuf=rbuf):
                sl_ = pl.ds(j * L, L)
                a = rbuf[2 * i, sl_]
                b = rbuf[2 * i + 1, sl_]
                obuf[i, sl_] = ga * a + gb * b
                return 0

            lax.fori_loop(0, D // L, body, 0)
        toff = wid * TPW + c * (CH // 2)
        pltpu.sync_copy(obuf, out_hbm.at[pl.ds(toff, CH // 2)])


def _combine(y, pos3, g_flat):
    mesh = plsc.VectorSubcoreMesh(core_axis_name="c", subcore_axis_name="s")
    return pl.kernel(
        _combine_body,
        out_type=jax.ShapeDtypeStruct((S, D), jnp.float32),
        mesh=mesh,
        scratch_types=[
            pltpu.VMEM((NCH, CH), jnp.int32),
            pltpu.VMEM((PPW,), jnp.float32),
            pltpu.VMEM((CH, D), jnp.float32),
            pltpu.VMEM((CH, D), jnp.float32),
            pltpu.VMEM((CH // 2, D), jnp.float32),
            pltpu.SemaphoreType.DMA,
            pltpu.SemaphoreType.DMA,
        ],
    )(y, pos3, g_flat)


def kernel(x, W_router, W1, b1, W2, b2):
    x2d = x.reshape(S, D)
    idx, gates = _router(x2d, W_router)

    pairs_e = idx.reshape(S * K)
    onehot = (pairs_e[:, None] == jnp.arange(E, dtype=jnp.int32)[None, :])
    cum = jnp.cumsum(onehot.astype(jnp.int32), axis=0)
    counts = cum[-1]
    nblk = (counts + T - 1) // T
    blk_base = jnp.concatenate(
        [jnp.zeros((1,), jnp.int32), jnp.cumsum(nblk)[:-1].astype(jnp.int32)])
    rank = jnp.take_along_axis(cum, pairs_e[:, None], axis=1)[:, 0] - 1
    pos = blk_base[pairs_e] * T + rank
    bids = jnp.arange(NB, dtype=jnp.int32)
    block_expert = (
        jnp.sum((blk_base[None, :] <= bids[:, None]).astype(jnp.int32), axis=1)
        - 1).astype(jnp.int32)
    block_valid = (bids < jnp.sum(nblk)).astype(jnp.int32)

    tok = jnp.arange(S * K, dtype=jnp.int32) // K
    tok3 = tok.reshape(NW, NCH, CH)
    pos3 = pos.reshape(NW, NCH, CH)

    x_sorted = _dispatch(x2d, tok3, pos3)
    y = _ffn(x_sorted, W1, b1, W2, b2, block_expert, block_valid)
    out = _combine(y, pos3, gates.reshape(S * K))
    return out.reshape(1, S, D)


# speedup vs baseline: 1.6904x; 1.3083x over previous
import functools

import jax
import jax.numpy as jnp
from jax import lax
from jax.experimental import pallas as pl
from jax.experimental.pallas import tpu as pltpu
from jax.experimental.pallas import tpu_sc as plsc

S, D, E, F, K = 2048, 1024, 8, 2048, 2
T = 128
NB = (S * K) // T + E
R = NB * T

NW = 32
PPW = (S * K) // NW
CH = 32
NCH = PPW // CH
TPW = S // NW
L = 16


def _router_body(x_ref, wr_ref, idx_ref, gate_ref):
    x = x_ref[...]
    logits = jnp.dot(x, wr_ref[...], preferred_element_type=jnp.float32)
    lane = lax.broadcasted_iota(jnp.int32, (S, E), 1)
    m1 = jnp.max(logits, axis=1, keepdims=True)
    i1 = jnp.argmax(logits, axis=1)[:, None]
    masked = jnp.where(lane == i1, -jnp.inf, logits)
    m2 = jnp.max(masked, axis=1, keepdims=True)
    i2 = jnp.argmax(masked, axis=1)[:, None]
    d = jnp.exp(m2 - m1)
    g1 = 1.0 / (1.0 + d)
    g2 = d / (1.0 + d)
    idx_ref[...] = jnp.concatenate([i1, i2], axis=1).astype(jnp.int32)
    gate_ref[...] = jnp.concatenate([g1, g2], axis=1)


def _router(x2d, wr):
    return pl.pallas_call(
        _router_body,
        out_shape=(
            jax.ShapeDtypeStruct((S, K), jnp.int32),
            jax.ShapeDtypeStruct((S, K), jnp.float32),
        ),
    )(x2d, wr)


def _dispatch_body(x_hbm, tok_hbm, pos_hbm, xs_hbm,
                   idx_t, idx_p, rows_a, rows_b, sg_a, sg_b, ss_a, ss_b):
    wid = lax.axis_index("s") * 2 + lax.axis_index("c")
    pltpu.sync_copy(tok_hbm.at[wid], idx_t)
    pltpu.sync_copy(pos_hbm.at[wid], idx_p)
    rows = (rows_a, rows_b)
    sg = (sg_a, sg_b)
    ss = (ss_a, ss_b)
    h_g = [None, None]
    h_s = [None, None]
    h_g[0] = pltpu.async_copy(x_hbm.at[idx_t.at[0]], rows[0], sg[0])
    for c in range(NCH):
        sl = c % 2
        if c + 1 < NCH:
            nsl = 1 - sl
            if h_s[nsl] is not None:
                h_s[nsl].wait()
            h_g[nsl] = pltpu.async_copy(
                x_hbm.at[idx_t.at[c + 1]], rows[nsl], sg[nsl])
        h_g[sl].wait()
        h_s[sl] = pltpu.async_copy(rows[sl], xs_hbm.at[idx_p.at[c]], ss[sl])
    h_s[0].wait()
    h_s[1].wait()


def _dispatch(x2d, tok3, pos3):
    mesh = plsc.VectorSubcoreMesh(core_axis_name="c", subcore_axis_name="s")
    return pl.kernel(
        _dispatch_body,
        out_type=jax.ShapeDtypeStruct((R, D), jnp.float32),
        mesh=mesh,
        scratch_types=[
            pltpu.VMEM((NCH, CH), jnp.int32),
            pltpu.VMEM((NCH, CH), jnp.int32),
            pltpu.VMEM((CH, D), jnp.float32),
            pltpu.VMEM((CH, D), jnp.float32),
            pltpu.SemaphoreType.DMA,
            pltpu.SemaphoreType.DMA,
            pltpu.SemaphoreType.DMA,
            pltpu.SemaphoreType.DMA,
        ],
    )(x2d, tok3, pos3)


def _ffn_body(be_ref, bv_ref, x_ref, w1_ref, b1_ref, w2_ref, b2_ref, o_ref):
    del be_ref
    b = pl.program_id(0)

    @pl.when(bv_ref[b] > 0)
    def _():
        h = jnp.dot(x_ref[...], w1_ref[0], preferred_element_type=jnp.float32)
        h = jax.nn.gelu(h + b1_ref[0])
        y = jnp.dot(h, w2_ref[0], preferred_element_type=jnp.float32)
        o_ref[...] = y + b2_ref[0]


def _ffn(x_sorted, W1, b1, W2, b2, block_expert, block_valid):
    grid_spec = pltpu.PrefetchScalarGridSpec(
        num_scalar_prefetch=2,
        grid=(NB,),
        in_specs=[
            pl.BlockSpec((T, D), lambda b, be, bv: (b, 0)),
            pl.BlockSpec((1, D, F), lambda b, be, bv: (be[b], 0, 0)),
            pl.BlockSpec((1, 1, F), lambda b, be, bv: (be[b], 0, 0)),
            pl.BlockSpec((1, F, D), lambda b, be, bv: (be[b], 0, 0)),
            pl.BlockSpec((1, 1, D), lambda b, be, bv: (be[b], 0, 0)),
        ],
        out_specs=pl.BlockSpec((T, D), lambda b, be, bv: (b, 0)),
    )
    return pl.pallas_call(
        _ffn_body,
        grid_spec=grid_spec,
        out_shape=jax.ShapeDtypeStruct((R, D), jnp.float32),
    )(block_expert, block_valid, x_sorted, W1,
      b1.reshape(E, 1, F), W2, b2.reshape(E, 1, D))


def _combine_body(y_hbm, pos_hbm, g_hbm, out_hbm,
                  idx_p, gall, rows_a, rows_b, obuf, sg_a, sg_b):
    wid = lax.axis_index("s") * 2 + lax.axis_index("c")
    pltpu.sync_copy(pos_hbm.at[wid], idx_p)
    pltpu.sync_copy(g_hbm.at[pl.ds(wid * PPW, PPW)], gall)
    rows = (rows_a, rows_b)
    sg = (sg_a, sg_b)
    h_g = [None, None]
    h_g[0] = pltpu.async_copy(y_hbm.at[idx_p.at[0]], rows[0], sg[0])
    for c in range(NCH):
        sl = c % 2
        if c + 1 < NCH:
            h_g[1 - sl] = pltpu.async_copy(
                y_hbm.at[idx_p.at[c + 1]], rows[1 - sl], sg[1 - sl])
        h_g[sl].wait()
        rbuf = rows[sl]
        gva = gall[pl.ds(c * CH, L)]
        gvb = gall[pl.ds(c * CH + L, L)]
        gs = ([gva[k] for k in range(L)] + [gvb[k] for k in range(L)])

        def body(j, _, gs=gs, rbuf=rbuf):
            sl_ = pl.ds(j * L, L)
            for i in range(CH // 2):
                a = rbuf[2 * i, sl_]
                b = rbuf[2 * i + 1, sl_]
                obuf[i, sl_] = gs[2 * i] * a + gs[2 * i + 1] * b
            return 0

        lax.fori_loop(0, D // L, body, 0)
        toff = wid * TPW + c * (CH // 2)
        pltpu.sync_copy(obuf, out_hbm.at[pl.ds(toff, CH // 2)])


def _combine(y, pos3, g_flat):
    mesh = plsc.VectorSubcoreMesh(core_axis_name="c", subcore_axis_name="s")
    return pl.kernel(
        _combine_body,
        out_type=jax.ShapeDtypeStruct((S, D), jnp.float32),
        mesh=mesh,
        scratch_types=[
            pltpu.VMEM((NCH, CH), jnp.int32),
            pltpu.VMEM((PPW,), jnp.float32),
            pltpu.VMEM((CH, D), jnp.float32),
            pltpu.VMEM((CH, D), jnp.float32),
            pltpu.VMEM((CH // 2, D), jnp.float32),
            pltpu.SemaphoreType.DMA,
            pltpu.SemaphoreType.DMA,
        ],
    )(y, pos3, g_flat)


def kernel(x, W_router, W1, b1, W2, b2):
    x2d = x.reshape(S, D)
    idx, gates = _router(x2d, W_router)

    pairs_e = idx.reshape(S * K)
    onehot = (pairs_e[:, None] == jnp.arange(E, dtype=jnp.int32)[None, :])
    cum = jnp.cumsum(onehot.astype(jnp.int32), axis=0)
    counts = cum[-1]
    nblk = (counts + T - 1) // T
    blk_base = jnp.concatenate(
        [jnp.zeros((1,), jnp.int32), jnp.cumsum(nblk)[:-1].astype(jnp.int32)])
    rank = jnp.take_along_axis(cum, pairs_e[:, None], axis=1)[:, 0] - 1
    pos = blk_base[pairs_e] * T + rank
    bids = jnp.arange(NB, dtype=jnp.int32)
    block_expert = (
        jnp.sum((blk_base[None, :] <= bids[:, None]).astype(jnp.int32), axis=1)
        - 1).astype(jnp.int32)
    block_valid = (bids < jnp.sum(nblk)).astype(jnp.int32)

    tok = jnp.arange(S * K, dtype=jnp.int32) // K
    tok3 = tok.reshape(NW, NCH, CH)
    pos3 = pos.reshape(NW, NCH, CH)

    x_sorted = _dispatch(x2d, tok3, pos3)
    y = _ffn(x_sorted, W1, b1, W2, b2, block_expert, block_valid)
    out = _combine(y, pos3, gates.reshape(S * K))
    return out.reshape(1, S, D)
